# Initial kernel scaffold; baseline (speedup 1.0000x reference)
import jax
import jax.numpy as jnp
from jax.experimental import pallas as pl


def kernel(xyz_f1, xyz_f2, W1, W2):
    raise NotImplementedError("write your pallas kernel here")


# speedup vs baseline: 4.7337x; 4.7337x over previous
import functools

import jax
import jax.numpy as jnp
from jax import lax
from jax.experimental import pallas as pl

STRIDE_H = 4
STRIDE_W = 8
K_NN = 32


def _pick_bq(n):
    best = 8
    for cand in range(8, 257, 8):
        if n % cand == 0:
            best = cand
    return best


def _knn_mlp_body(q_ref, k_ref, w1_ref, w2_ref, o_ref, *, n_keys):
    qb = q_ref[0]
    kb = k_ref[0]
    w1 = w1_ref[...]
    w2 = w2_ref[...]
    bq = qb.shape[0]

    qq = jnp.sum(qb * qb, axis=-1, keepdims=True)
    kk = jnp.sum(kb * kb, axis=-1)[None, :]
    d2 = qq - 2.0 * jnp.dot(qb, kb.T,
                            preferred_element_type=jnp.float32) + kk

    kx = kb[:, 0][None, :]
    ky = kb[:, 1][None, :]
    kz = kb[:, 2][None, :]
    iota = lax.broadcasted_iota(jnp.int32, (bq, n_keys), 1)
    big_i = jnp.int32(2**30)
    inf = jnp.float32(jnp.inf)

    out = jnp.full((bq, 64), -inf, dtype=jnp.float32)
    for _ in range(K_NN):
        m = jnp.min(d2, axis=1, keepdims=True)
        at_min = d2 == m
        idx = jnp.min(jnp.where(at_min, iota, big_i), axis=1,
                      keepdims=True)
        onehot = iota == idx
        px = jnp.sum(jnp.where(onehot, kx, 0.0), axis=1, keepdims=True)
        py = jnp.sum(jnp.where(onehot, ky, 0.0), axis=1, keepdims=True)
        pz = jnp.sum(jnp.where(onehot, kz, 0.0), axis=1, keepdims=True)
        d2 = jnp.where(onehot, inf, d2)

        p = jnp.concatenate([px, py, pz], axis=1)
        feat = jnp.concatenate([p - qb, p], axis=1)
        h = jnp.maximum(jnp.dot(feat, w1,
                                preferred_element_type=jnp.float32), 0.0)
        h = jnp.maximum(jnp.dot(h, w2,
                                preferred_element_type=jnp.float32), 0.0)
        out = jnp.maximum(out, h)

    o_ref[0] = out


def kernel(xyz_f1, xyz_f2, W1, W2):
    B = xyz_f1.shape[0]
    q = xyz_f1[:, ::STRIDE_H, ::STRIDE_W, :].reshape(B, -1, 3)
    k = xyz_f2[:, ::STRIDE_H, ::STRIDE_W, :].reshape(B, -1, 3)
    n = q.shape[1]
    bq = _pick_bq(n)

    grid = (B, n // bq)
    body = functools.partial(_knn_mlp_body, n_keys=n)
    return pl.pallas_call(
        body,
        grid=grid,
        in_specs=[
            pl.BlockSpec((1, bq, 3), lambda b, i: (b, i, 0)),
            pl.BlockSpec((1, n, 3), lambda b, i: (b, 0, 0)),
            pl.BlockSpec((6, 128), lambda b, i: (0, 0)),
            pl.BlockSpec((128, 64), lambda b, i: (0, 0)),
        ],
        out_specs=pl.BlockSpec((1, bq, 64), lambda b, i: (b, i, 0)),
        out_shape=jax.ShapeDtypeStruct((B, n, 64), jnp.float32),
    )(q, k, W1, W2)


# speedup vs baseline: 5.3507x; 1.1303x over previous
import functools

import jax
import jax.numpy as jnp
from jax import lax
from jax.experimental import pallas as pl

STRIDE_H = 4
STRIDE_W = 8
K_NN = 32


def _pick_bq(n):
    best = 8
    for cand in range(8, 257, 8):
        if n % cand == 0:
            best = cand
    return best


def _knn_mlp_body(q_ref, k_ref, w1_ref, w2_ref, o_ref, *, n_keys):
    qb = q_ref[0]
    kb = k_ref[0]
    w1 = w1_ref[...]
    w2 = w2_ref[...]
    bq = qb.shape[0]

    w1a = w1[0:3]
    w1s = w1[0:3] + w1[3:6]
    vq = jnp.dot(qb, w1a, preferred_element_type=jnp.float32)

    qq = jnp.sum(qb * qb, axis=-1, keepdims=True)
    kk = jnp.sum(kb * kb, axis=-1)[None, :]
    d2 = qq - 2.0 * jnp.dot(qb, kb.T,
                            preferred_element_type=jnp.float32) + kk

    iota = lax.broadcasted_iota(jnp.int32, (bq, n_keys), 1)
    big_i = jnp.int32(2**30)
    inf = jnp.float32(jnp.inf)

    out = jnp.full((bq, 64), -inf, dtype=jnp.float32)
    for _ in range(K_NN):
        m = jnp.min(d2, axis=1, keepdims=True)
        idx = jnp.min(jnp.where(d2 == m, iota, big_i), axis=1,
                      keepdims=True)
        oh = iota == idx
        ohf = oh.astype(jnp.float32)
        d2 = jnp.where(oh, inf, d2)
        p = jnp.dot(ohf, kb, preferred_element_type=jnp.float32)
        h = jnp.maximum(jnp.dot(p, w1s,
                                preferred_element_type=jnp.float32) - vq, 0.0)
        h = jnp.maximum(jnp.dot(h, w2,
                                preferred_element_type=jnp.float32), 0.0)
        out = jnp.maximum(out, h)

    o_ref[0] = out


def kernel(xyz_f1, xyz_f2, W1, W2):
    B = xyz_f1.shape[0]
    q = xyz_f1[:, ::STRIDE_H, ::STRIDE_W, :].reshape(B, -1, 3)
    k = xyz_f2[:, ::STRIDE_H, ::STRIDE_W, :].reshape(B, -1, 3)
    n = q.shape[1]
    bq = _pick_bq(n)

    grid = (B, n // bq)
    body = functools.partial(_knn_mlp_body, n_keys=n)
    return pl.pallas_call(
        body,
        grid=grid,
        in_specs=[
            pl.BlockSpec((1, bq, 3), lambda b, i: (b, i, 0)),
            pl.BlockSpec((1, n, 3), lambda b, i: (b, 0, 0)),
            pl.BlockSpec((6, 128), lambda b, i: (0, 0)),
            pl.BlockSpec((128, 64), lambda b, i: (0, 0)),
        ],
        out_specs=pl.BlockSpec((1, bq, 64), lambda b, i: (b, i, 0)),
        out_shape=jax.ShapeDtypeStruct((B, n, 64), jnp.float32),
    )(q, k, W1, W2)


# speedup vs baseline: 6.1488x; 1.1492x over previous
import functools

import jax
import jax.numpy as jnp
from jax import lax
from jax.experimental import pallas as pl

STRIDE_H = 4
STRIDE_W = 8
K_NN = 32


def _pick_bq(n):
    best = 8
    for cand in range(8, 721, 8):
        if n % cand == 0:
            best = cand
    return best


def _knn_mlp_body(q_ref, k_ref, w1_ref, w2_ref, o_ref, *, n_keys):
    qb = q_ref[0]
    kb = k_ref[0]
    w1 = w1_ref[...]
    w2 = w2_ref[...]
    bq = qb.shape[0]

    w1a = w1[0:3]
    w1s = w1[0:3] + w1[3:6]
    vq = jnp.dot(qb, w1a, preferred_element_type=jnp.float32)

    qq = jnp.sum(qb * qb, axis=-1, keepdims=True)
    kk = jnp.sum(kb * kb, axis=-1)[None, :]
    d2 = qq - 2.0 * jnp.dot(qb, kb.T,
                            preferred_element_type=jnp.float32) + kk

    iota = lax.broadcasted_iota(jnp.int32, (bq, n_keys), 1)
    big_i = jnp.int32(2**30)
    inf = jnp.float32(jnp.inf)

    out = jnp.full((bq, 64), -inf, dtype=jnp.float32)
    m = jnp.min(d2, axis=1, keepdims=True)
    for _ in range(K_NN):
        idx = jnp.min(jnp.where(d2 == m, iota, big_i), axis=1,
                      keepdims=True)
        oh = iota == idx
        ohf = oh.astype(jnp.float32)
        d2 = jnp.where(oh, inf, d2)
        m = jnp.min(d2, axis=1, keepdims=True)
        p = jnp.dot(ohf, kb, preferred_element_type=jnp.float32)
        h = jnp.maximum(jnp.dot(p, w1s,
                                preferred_element_type=jnp.float32) - vq, 0.0)
        h = jnp.maximum(jnp.dot(h, w2,
                                preferred_element_type=jnp.float32), 0.0)
        out = jnp.maximum(out, h)

    o_ref[0] = out


def kernel(xyz_f1, xyz_f2, W1, W2):
    B = xyz_f1.shape[0]
    q = xyz_f1[:, ::STRIDE_H, ::STRIDE_W, :].reshape(B, -1, 3)
    k = xyz_f2[:, ::STRIDE_H, ::STRIDE_W, :].reshape(B, -1, 3)
    n = q.shape[1]
    bq = _pick_bq(n)

    grid = (B, n // bq)
    body = functools.partial(_knn_mlp_body, n_keys=n)
    return pl.pallas_call(
        body,
        grid=grid,
        in_specs=[
            pl.BlockSpec((1, bq, 3), lambda b, i: (b, i, 0)),
            pl.BlockSpec((1, n, 3), lambda b, i: (b, 0, 0)),
            pl.BlockSpec((6, 128), lambda b, i: (0, 0)),
            pl.BlockSpec((128, 64), lambda b, i: (0, 0)),
        ],
        out_specs=pl.BlockSpec((1, bq, 64), lambda b, i: (b, i, 0)),
        out_shape=jax.ShapeDtypeStruct((B, n, 64), jnp.float32),
    )(q, k, W1, W2)


# speedup vs baseline: 6.6424x; 1.0803x over previous
import jax
import jax.numpy as jnp
from jax import lax
from jax.experimental import pallas as pl
from jax.experimental.pallas import tpu as pltpu
from jax.experimental.pallas import tpu_sc as plsc

STRIDE_H = 4
STRIDE_W = 8
K_NN = 32

N_Q = 3600
N_B = 8
ROWS_PER_SUBCORE = N_B * N_Q // 32
SEG = 225
N_SEGS = ROWS_PER_SUBCORE // SEG
N_CHUNKS = N_Q // 16


def _sc_topk_kernel(d2m, kx, ky, kz, qx, qy, qz, out,
                    kxv, kyv, kzv, qxv, qyv, qzv, rbuf, obuf):
    cid = lax.axis_index("c")
    sid = lax.axis_index("s")
    wid = sid * 2 + cid
    batch = wid // 4
    qbase = (wid % 4) * ROWS_PER_SUBCORE

    bsl = pl.ds(batch * N_Q, N_Q)
    pltpu.sync_copy(kx.at[bsl], kxv)
    pltpu.sync_copy(ky.at[bsl], kyv)
    pltpu.sync_copy(kz.at[bsl], kzv)
    pltpu.sync_copy(qx.at[bsl], qxv.at[pl.ds(0, N_Q)])
    pltpu.sync_copy(qy.at[bsl], qyv.at[pl.ds(0, N_Q)])
    pltpu.sync_copy(qz.at[bsl], qzv.at[pl.ds(0, N_Q)])

    iota16 = lax.iota(jnp.int32, 16)
    inf = jnp.float32(jnp.inf)

    def _row_body(r, seg):
        row = qbase + seg * SEG + r
        g = wid * ROWS_PER_SUBCORE + seg * SEG + r
        pltpu.sync_copy(d2m.at[pl.ds(g * N_Q, N_Q)], rbuf)
        qxs = qxv[pl.ds(row, 16)][0]
        qys = qyv[pl.ds(row, 16)][0]
        qzs = qzv[pl.ds(row, 16)][0]
        qbx = jnp.full((16,), qxs, jnp.float32)
        qby = jnp.full((16,), qys, jnp.float32)
        qbz = jnp.full((16,), qzs, jnp.float32)

        def _chunk_body(c, carry):
            t0k, t0i, t1k, t1i, thr = carry
            d2 = rbuf[pl.ds(c * 16, 16)]
            idxc = c * 16 + iota16

            sk, si = plsc.sort_key_val(d2, idxc)

            def _merge(args):
                t0k, t0i, t1k, t1i, _ = args
                rk = lax.rev(sk, (0,))
                ri = lax.rev(si, (0,))
                le = t1k <= rk
                n1 = jnp.where(le, t1k, rk)
                n1i = jnp.where(le, t1i, ri)
                le0 = t0k <= n1
                m0 = jnp.where(le0, t0k, n1)
                m0i = jnp.where(le0, t0i, n1i)
                m1 = jnp.where(le0, n1, t0k)
                m1i = jnp.where(le0, n1i, t0i)
                a0k, a0i = plsc.sort_key_val(m0, m0i)
                a1k, a1i = plsc.sort_key_val(m1, m1i)
                return a0k, a0i, a1k, a1i, a1k[15]

            hit = sk[0] < thr
            return lax.cond(hit, _merge, lambda a: a,
                            (t0k, t0i, t1k, t1i, thr))

        init = (jnp.full((16,), inf, jnp.float32), jnp.zeros((16,), jnp.int32),
                jnp.full((16,), inf, jnp.float32), jnp.zeros((16,), jnp.int32),
                inf)
        t0k, t0i, t1k, t1i, _ = lax.fori_loop(0, N_CHUNKS, _chunk_body, init)

        p0x = plsc.load_gather(kxv, [t0i])
        p0y = plsc.load_gather(kyv, [t0i])
        p0z = plsc.load_gather(kzv, [t0i])
        p1x = plsc.load_gather(kxv, [t1i])
        p1y = plsc.load_gather(kyv, [t1i])
        p1z = plsc.load_gather(kzv, [t1i])

        off0 = r * 256 + iota16 * 8
        off1 = off0 + 128

        def put(base, col, vals):
            plsc.store_scatter(obuf, [base + col], vals)

        put(off0, 0, p0x - qbx)
        put(off0, 1, p0y - qby)
        put(off0, 2, p0z - qbz)
        put(off0, 3, p0x)
        put(off0, 4, p0y)
        put(off0, 5, p0z)
        put(off1, 0, p1x - qbx)
        put(off1, 1, p1y - qby)
        put(off1, 2, p1z - qbz)
        put(off1, 3, p1x)
        put(off1, 4, p1y)
        put(off1, 5, p1z)
        zeros = jnp.zeros((16,), jnp.float32)
        put(off0, 6, zeros)
        put(off0, 7, zeros)
        put(off1, 6, zeros)
        put(off1, 7, zeros)
        return seg

    def _seg_body(seg, carry):
        lax.fori_loop(0, SEG, _row_body, seg)
        off = (wid * ROWS_PER_SUBCORE + seg * SEG) * 256
        pltpu.sync_copy(obuf, out.at[pl.ds(off, SEG * 256)])
        return carry

    lax.fori_loop(0, N_SEGS, _seg_body, 0)


def _d2_body(q_ref, k_ref, o_ref):
    qb = q_ref[0]
    kb = k_ref[0]
    qq = jnp.sum(qb * qb, axis=-1, keepdims=True)
    kk = jnp.sum(kb * kb, axis=-1)[None, :]
    o_ref[0] = qq - 2.0 * jnp.dot(qb, kb.T,
                                  preferred_element_type=jnp.float32) + kk


def _mlp_body(f_ref, w1_ref, w2_ref, o_ref):
    f = f_ref[...]
    w1 = w1_ref[...]
    w2 = w2_ref[...]
    out = jnp.full((f.shape[0], 64), -jnp.inf, dtype=jnp.float32)
    for j in range(K_NN):
        fj = f[:, j * 8:(j + 1) * 8]
        h = jnp.maximum(jnp.dot(fj, w1,
                                preferred_element_type=jnp.float32), 0.0)
        h = jnp.maximum(jnp.dot(h, w2,
                                preferred_element_type=jnp.float32), 0.0)
        out = jnp.maximum(out, h)
    o_ref[...] = out


def kernel(xyz_f1, xyz_f2, W1, W2):
    B = xyz_f1.shape[0]
    q = xyz_f1[:, ::STRIDE_H, ::STRIDE_W, :].reshape(B, -1, 3)
    k = xyz_f2[:, ::STRIDE_H, ::STRIDE_W, :].reshape(B, -1, 3)
    n = q.shape[1]

    qx, qy, qz = [q[..., i].reshape(-1) for i in range(3)]
    kx, ky, kz = [k[..., i].reshape(-1) for i in range(3)]

    bq = 720
    d2m = pl.pallas_call(
        _d2_body,
        grid=(B, n // bq),
        in_specs=[
            pl.BlockSpec((1, bq, 3), lambda b, i: (b, i, 0)),
            pl.BlockSpec((1, n, 3), lambda b, i: (b, 0, 0)),
        ],
        out_specs=pl.BlockSpec((1, bq, n), lambda b, i: (b, i, 0)),
        out_shape=jax.ShapeDtypeStruct((B, n, n), jnp.float32),
    )(q, k).reshape(-1)

    mesh = plsc.VectorSubcoreMesh(core_axis_name="c", subcore_axis_name="s")
    sc_fn = pl.kernel(
        _sc_topk_kernel, mesh=mesh,
        compiler_params=pltpu.CompilerParams(needs_layout_passes=False),
        out_type=jax.ShapeDtypeStruct((B * n * K_NN * 8,), jnp.float32),
        scratch_types=[
            pltpu.VMEM((n,), jnp.float32),
            pltpu.VMEM((n,), jnp.float32),
            pltpu.VMEM((n,), jnp.float32),
            pltpu.VMEM((n + 16,), jnp.float32),
            pltpu.VMEM((n + 16,), jnp.float32),
            pltpu.VMEM((n + 16,), jnp.float32),
            pltpu.VMEM((n,), jnp.float32),
            pltpu.VMEM((SEG * K_NN * 8,), jnp.float32),
        ],
    )
    feat = sc_fn(d2m, kx, ky, kz, qx, qy, qz).reshape(B * n, K_NN * 8)

    w1p = jnp.concatenate(
        [W1, jnp.zeros((2, W1.shape[1]), W1.dtype)], axis=0)

    tbq = 960
    grid = (B * n // tbq,)
    out = pl.pallas_call(
        _mlp_body,
        grid=grid,
        in_specs=[
            pl.BlockSpec((tbq, K_NN * 8), lambda i: (i, 0)),
            pl.BlockSpec((8, 128), lambda i: (0, 0)),
            pl.BlockSpec((128, 64), lambda i: (0, 0)),
        ],
        out_specs=pl.BlockSpec((tbq, 64), lambda i: (i, 0)),
        out_shape=jax.ShapeDtypeStruct((B * n, 64), jnp.float32),
    )(feat, w1p, W2)
    return out.reshape(B, n, 64)


# speedup vs baseline: 7.3677x; 1.1092x over previous
import jax
import jax.numpy as jnp
from jax import lax
from jax.experimental import pallas as pl
from jax.experimental.pallas import tpu as pltpu
from jax.experimental.pallas import tpu_sc as plsc

STRIDE_H = 4
STRIDE_W = 8
K_NN = 32

N_Q = 3600
N_B = 8
ROWS_PER_SUBCORE = N_B * N_Q // 32
SEG = 150
N_SEGS = ROWS_PER_SUBCORE // SEG
N_CHUNKS = N_Q // 16


def _sc_topk_kernel(d2m, kx, ky, kz, qx, qy, qz, out,
                    kxv, kyv, kzv, qxv, qyv, qzv, rbuf0, rbuf1, obuf, sem0, sem1):
    cid = lax.axis_index("c")
    sid = lax.axis_index("s")
    wid = sid * 2 + cid
    batch = wid // 4
    qbase = (wid % 4) * ROWS_PER_SUBCORE

    bsl = pl.ds(batch * N_Q, N_Q)
    pltpu.sync_copy(kx.at[bsl], kxv)
    pltpu.sync_copy(ky.at[bsl], kyv)
    pltpu.sync_copy(kz.at[bsl], kzv)
    pltpu.sync_copy(qx.at[bsl], qxv.at[pl.ds(0, N_Q)])
    pltpu.sync_copy(qy.at[bsl], qyv.at[pl.ds(0, N_Q)])
    pltpu.sync_copy(qz.at[bsl], qzv.at[pl.ds(0, N_Q)])

    iota16 = lax.iota(jnp.int32, 16)
    inf = jnp.float32(jnp.inf)

    def _process_row(r, seg, buf):
        row = qbase + seg * SEG + r
        qxs = qxv[pl.ds(row, 16)][0]
        qys = qyv[pl.ds(row, 16)][0]
        qzs = qzv[pl.ds(row, 16)][0]
        qbx = jnp.full((16,), qxs, jnp.float32)
        qby = jnp.full((16,), qys, jnp.float32)
        qbz = jnp.full((16,), qzs, jnp.float32)

        def _chunk_body(c, carry):
            t0k, t0i, t1k, t1i, thr = carry
            d2 = buf[pl.ds(c * 16, 16)]
            idxc = c * 16 + iota16

            sk, si = plsc.sort_key_val(d2, idxc)

            def _merge(args):
                t0k, t0i, t1k, t1i, _ = args
                rk = lax.rev(sk, (0,))
                ri = lax.rev(si, (0,))
                le = t1k <= rk
                n1 = jnp.where(le, t1k, rk)
                n1i = jnp.where(le, t1i, ri)
                le0 = t0k <= n1
                m0 = jnp.where(le0, t0k, n1)
                m0i = jnp.where(le0, t0i, n1i)
                m1 = jnp.where(le0, n1, t0k)
                m1i = jnp.where(le0, n1i, t0i)
                a0k, a0i = plsc.sort_key_val(m0, m0i)
                a1k, a1i = plsc.sort_key_val(m1, m1i)
                return a0k, a0i, a1k, a1i, a1k[15]

            hit = sk[0] < thr
            return lax.cond(hit, _merge, lambda a: a,
                            (t0k, t0i, t1k, t1i, thr))

        init = (jnp.full((16,), inf, jnp.float32), jnp.zeros((16,), jnp.int32),
                jnp.full((16,), inf, jnp.float32), jnp.zeros((16,), jnp.int32),
                inf)
        t0k, t0i, t1k, t1i, _ = lax.fori_loop(0, N_CHUNKS, _chunk_body, init)

        p0x = plsc.load_gather(kxv, [t0i])
        p0y = plsc.load_gather(kyv, [t0i])
        p0z = plsc.load_gather(kzv, [t0i])
        p1x = plsc.load_gather(kxv, [t1i])
        p1y = plsc.load_gather(kyv, [t1i])
        p1z = plsc.load_gather(kzv, [t1i])

        off0 = r * 256 + iota16 * 8
        off1 = off0 + 128

        def put(base, col, vals):
            plsc.store_scatter(obuf, [base + col], vals)

        put(off0, 0, p0x - qbx)
        put(off0, 1, p0y - qby)
        put(off0, 2, p0z - qbz)
        put(off0, 3, p0x)
        put(off0, 4, p0y)
        put(off0, 5, p0z)
        put(off1, 0, p1x - qbx)
        put(off1, 1, p1y - qby)
        put(off1, 2, p1z - qbz)
        put(off1, 3, p1x)
        put(off1, 4, p1y)
        put(off1, 5, p1z)
        zeros = jnp.zeros((16,), jnp.float32)
        put(off0, 6, zeros)
        put(off0, 7, zeros)
        put(off1, 6, zeros)
        put(off1, 7, zeros)

    def _seg_body(seg, carry):
        g0 = wid * ROWS_PER_SUBCORE + seg * SEG

        def rsl(r):
            return pl.ds((g0 + r) * N_Q, N_Q)

        pltpu.async_copy(d2m.at[rsl(0)], rbuf0, sem0)
        pltpu.async_copy(d2m.at[rsl(1)], rbuf1, sem1)

        def _pair_body(i, carry2):
            r0 = 2 * i
            r1 = 2 * i + 1
            pltpu.make_async_copy(d2m.at[rsl(r0)], rbuf0, sem0).wait()
            _process_row(r0, seg, rbuf0)
            pltpu.async_copy(d2m.at[rsl(jnp.minimum(r0 + 2, SEG - 2))],
                             rbuf0, sem0)
            pltpu.make_async_copy(d2m.at[rsl(r1)], rbuf1, sem1).wait()
            _process_row(r1, seg, rbuf1)
            pltpu.async_copy(d2m.at[rsl(jnp.minimum(r1 + 2, SEG - 1))],
                             rbuf1, sem1)
            return carry2

        lax.fori_loop(0, SEG // 2, _pair_body, 0)
        pltpu.make_async_copy(d2m.at[rsl(SEG - 2)], rbuf0, sem0).wait()
        pltpu.make_async_copy(d2m.at[rsl(SEG - 1)], rbuf1, sem1).wait()
        off = g0 * 256
        pltpu.sync_copy(obuf, out.at[pl.ds(off, SEG * 256)])
        return carry

    lax.fori_loop(0, N_SEGS, _seg_body, 0)


def _d2_body(q_ref, k_ref, o_ref):
    qb = q_ref[0]
    kb = k_ref[0]
    qq = jnp.sum(qb * qb, axis=-1, keepdims=True)
    kk = jnp.sum(kb * kb, axis=-1)[None, :]
    o_ref[0] = qq - 2.0 * jnp.dot(qb, kb.T,
                                  preferred_element_type=jnp.float32) + kk


def _mlp_body(f_ref, w1_ref, w2_ref, o_ref):
    f = f_ref[...]
    w1 = w1_ref[...]
    w2 = w2_ref[...]
    out = jnp.full((f.shape[0], 64), -jnp.inf, dtype=jnp.float32)
    for j in range(K_NN):
        fj = f[:, j * 8:(j + 1) * 8]
        h = jnp.maximum(jnp.dot(fj, w1,
                                preferred_element_type=jnp.float32), 0.0)
        h = jnp.maximum(jnp.dot(h, w2,
                                preferred_element_type=jnp.float32), 0.0)
        out = jnp.maximum(out, h)
    o_ref[...] = out


def kernel(xyz_f1, xyz_f2, W1, W2):
    B = xyz_f1.shape[0]
    q = xyz_f1[:, ::STRIDE_H, ::STRIDE_W, :].reshape(B, -1, 3)
    k = xyz_f2[:, ::STRIDE_H, ::STRIDE_W, :].reshape(B, -1, 3)
    n = q.shape[1]

    qx, qy, qz = [q[..., i].reshape(-1) for i in range(3)]
    kx, ky, kz = [k[..., i].reshape(-1) for i in range(3)]

    bq = 720
    d2m = pl.pallas_call(
        _d2_body,
        grid=(B, n // bq),
        in_specs=[
            pl.BlockSpec((1, bq, 3), lambda b, i: (b, i, 0)),
            pl.BlockSpec((1, n, 3), lambda b, i: (b, 0, 0)),
        ],
        out_specs=pl.BlockSpec((1, bq, n), lambda b, i: (b, i, 0)),
        out_shape=jax.ShapeDtypeStruct((B, n, n), jnp.float32),
    )(q, k).reshape(-1)

    mesh = plsc.VectorSubcoreMesh(core_axis_name="c", subcore_axis_name="s")
    sc_fn = pl.kernel(
        _sc_topk_kernel, mesh=mesh,
        compiler_params=pltpu.CompilerParams(needs_layout_passes=False),
        out_type=jax.ShapeDtypeStruct((B * n * K_NN * 8,), jnp.float32),
        scratch_types=[
            pltpu.VMEM((n,), jnp.float32),
            pltpu.VMEM((n,), jnp.float32),
            pltpu.VMEM((n,), jnp.float32),
            pltpu.VMEM((n + 16,), jnp.float32),
            pltpu.VMEM((n + 16,), jnp.float32),
            pltpu.VMEM((n + 16,), jnp.float32),
            pltpu.VMEM((n,), jnp.float32),
            pltpu.VMEM((n,), jnp.float32),
            pltpu.VMEM((SEG * K_NN * 8,), jnp.float32),
            pltpu.SemaphoreType.DMA,
            pltpu.SemaphoreType.DMA,
        ],
    )
    feat = sc_fn(d2m, kx, ky, kz, qx, qy, qz).reshape(B * n, K_NN * 8)

    w1p = jnp.concatenate(
        [W1, jnp.zeros((2, W1.shape[1]), W1.dtype)], axis=0)

    tbq = 960
    grid = (B * n // tbq,)
    out = pl.pallas_call(
        _mlp_body,
        grid=grid,
        in_specs=[
            pl.BlockSpec((tbq, K_NN * 8), lambda i: (i, 0)),
            pl.BlockSpec((8, 128), lambda i: (0, 0)),
            pl.BlockSpec((128, 64), lambda i: (0, 0)),
        ],
        out_specs=pl.BlockSpec((tbq, 64), lambda i: (i, 0)),
        out_shape=jax.ShapeDtypeStruct((B * n, 64), jnp.float32),
    )(feat, w1p, W2)
    return out.reshape(B, n, 64)
